# baseline (speedup 1.0000x reference)
import functools

import jax
import jax.numpy as jnp
from jax import lax
from jax.experimental import pallas as pl
from jax.experimental.pallas import tpu as pltpu
from jax.experimental.pallas import tpu_sc as plsc

N = 10000
E = 320000
D = 128
H = 2
AH = 16

NC = 2
NS = 16
NW = NC * NS
EPW = E // NW
B = 80
NB = EPW // B
N2 = 10240
RPT = N2 // NS
BE = 8000


def _stage_a(x, W1, b1, W2, b2, A1a, a1v, A1b):
    BN = 1000
    grid = (H, N // BN)

    def body(x_ref, w1_ref, b1_ref, w2_ref, b2_ref, a1a_ref, a1_ref,
             a1b_ref, feat_ref, ps_ref, pd_ref):
        xb = x_ref[...]
        f1 = jnp.maximum(
            jnp.dot(xb, w1_ref[0], preferred_element_type=jnp.float32)
            + b1_ref[0], 0.0)
        ft = (jnp.dot(f1, w2_ref[0], preferred_element_type=jnp.float32)
              + b2_ref[0])
        feat_ref[0] = ft
        ps_ref[0] = jnp.dot(ft, a1a_ref[0], preferred_element_type=jnp.float32)
        pd_ref[0] = (jnp.dot(ft, a1b_ref[0], preferred_element_type=jnp.float32)
                     + a1_ref[0])

    return pl.pallas_call(
        body,
        grid=grid,
        in_specs=[
            pl.BlockSpec((BN, D), lambda h, i: (i, 0)),
            pl.BlockSpec((1, D, D), lambda h, i: (h, 0, 0)),
            pl.BlockSpec((1, 1, D), lambda h, i: (h, 0, 0)),
            pl.BlockSpec((1, D, D), lambda h, i: (h, 0, 0)),
            pl.BlockSpec((1, 1, D), lambda h, i: (h, 0, 0)),
            pl.BlockSpec((1, D, AH), lambda h, i: (h, 0, 0)),
            pl.BlockSpec((1, 1, AH), lambda h, i: (h, 0, 0)),
            pl.BlockSpec((1, D, AH), lambda h, i: (h, 0, 0)),
        ],
        out_specs=[
            pl.BlockSpec((1, BN, D), lambda h, i: (h, i, 0)),
            pl.BlockSpec((1, BN, AH), lambda h, i: (h, i, 0)),
            pl.BlockSpec((1, BN, AH), lambda h, i: (h, i, 0)),
        ],
        out_shape=[
            jax.ShapeDtypeStruct((H, N, D), jnp.float32),
            jax.ShapeDtypeStruct((H, N, AH), jnp.float32),
            jax.ShapeDtypeStruct((H, N, AH), jnp.float32),
        ],
    )(x, W1, b1[:, None, :], W2, b2[:, None, :], A1a, a1v[:, None, :], A1b)


def _b1_body(ps0, pd0, ps1, pd1, srcA, dstA,
             qr0, qr1,
             srcb0, dstb0, g10, g20,
             srcb1, dstb1, g11, g21,
             lsem0, lsem1, gsem0, gsem1, wsem0, wsem1):
    c = lax.axis_index("c")
    s = lax.axis_index("s")
    wid = s * NC + c
    ebase = wid * EPW3

    sets = ((srcb0, dstb0, g10, g20, lsem0, gsem0, wsem0),
            (srcb1, dstb1, g11, g21, lsem1, gsem1, wsem1))

    def issue_ld(b, p):
        srcb, dstb, _, _, lsem, _, _ = sets[p]
        base = ebase + b * B3B
        pltpu.async_copy(srcA.at[pl.ds(base, B3B)], srcb, lsem)
        pltpu.async_copy(dstA.at[pl.ds(base, B3B)], dstb, lsem)

    def drain_ld(p):
        srcb, dstb, _, _, lsem, _, _ = sets[p]
        pltpu.make_async_copy(srcA.at[pl.ds(0, B3B)], srcb, lsem).wait()
        pltpu.make_async_copy(dstA.at[pl.ds(0, B3B)], dstb, lsem).wait()

    def issue_gather(p, ps_h, pd_h):
        srcb, dstb, g1, g2, _, gsem, _ = sets[p]
        pltpu.async_copy(ps_h.at[srcb], g1, gsem)
        pltpu.async_copy(pd_h.at[dstb], g2, gsem)

    def drain_gather(p, ps_h):
        _, _, g1, g2, _, gsem, _ = sets[p]
        pltpu.make_async_copy(ps_h.at[pl.ds(0, B3B)], g1, gsem).wait()
        pltpu.make_async_copy(ps_h.at[pl.ds(0, B3B)], g2, gsem).wait()

    def drain_write(p, qr_h):
        _, _, g1, _, _, _, wsem = sets[p]
        pltpu.make_async_copy(g1, qr_h.at[pl.ds(0, B3B)], wsem).wait()

    def add_write(b, p, qr_h):
        _, _, g1, g2, _, _, wsem = sets[p]
        def add_e(e, cy):
            g1[e, :] = g1[e, :] + g2[e, :]
            return cy
        lax.fori_loop(0, B3B, add_e, 0)
        pltpu.async_copy(g1, qr_h.at[pl.ds(ebase + b * B3B, B3B)], wsem)

    for ps_h, pd_h, qr_h in ((ps0, pd0, qr0), (ps1, pd1, qr1)):
        issue_ld(0, 0)
        drain_ld(0)
        issue_gather(0, ps_h, pd_h)
        issue_ld(1, 1)

        def it(t, carry, ps_h=ps_h, pd_h=pd_h, qr_h=qr_h):
            drain_ld(1)
            @pl.when(t > 0)
            def _():
                drain_write(1, qr_h)
            issue_gather(1, ps_h, pd_h)
            drain_gather(0, ps_h)
            add_write(2 * t, 0, qr_h)
            @pl.when(t < NT - 1)
            def _():
                issue_ld(2 * (t + 1), 0)
            @pl.when(t < NT - 1)
            def _():
                drain_ld(0)
                drain_write(0, qr_h)
                issue_gather(0, ps_h, pd_h)
            drain_gather(1, ps_h)
            add_write(2 * t + 1, 1, qr_h)
            @pl.when(t < NT - 1)
            def _():
                issue_ld(2 * (t + 1) + 1, 1)
            return carry
        lax.fori_loop(0, NT, it, 0)
        drain_write(0, qr_h)
        drain_write(1, qr_h)

        @pl.when(wid < NEXT)
        def _(ps_h=ps_h, pd_h=pd_h, qr_h=qr_h):
            tb = EXT_BASE + wid * B3B
            pltpu.sync_copy(srcA.at[pl.ds(tb, B3B)], srcb0)
            pltpu.sync_copy(dstA.at[pl.ds(tb, B3B)], dstb0)
            cp1 = pltpu.async_copy(ps_h.at[srcb0], g10, gsem0)
            cp2 = pltpu.async_copy(pd_h.at[dstb0], g20, gsem0)
            cp1.wait()
            cp2.wait()
            def add_e(e, cy):
                g10[e, :] = g10[e, :] + g20[e, :]
                return cy
            lax.fori_loop(0, B3B, add_e, 0)
            pltpu.sync_copy(g10, qr_h.at[pl.ds(tb, B3B)])


def _stage_b1(ps, pd, src, dst):
    mesh = plsc.VectorSubcoreMesh(core_axis_name="c", subcore_axis_name="s")
    kfn = functools.partial(
        pl.kernel,
        mesh=mesh,
        compiler_params=pltpu.CompilerParams(use_tc_tiling_on_sc=False),
        out_type=[jax.ShapeDtypeStruct((E, AH), jnp.float32)] * 2,
        scratch_types=[
            pltpu.VMEM((B3B,), jnp.int32),
            pltpu.VMEM((B3B,), jnp.int32),
            pltpu.VMEM((B3B, AH), jnp.float32),
            pltpu.VMEM((B3B, AH), jnp.float32),
            pltpu.VMEM((B3B,), jnp.int32),
            pltpu.VMEM((B3B,), jnp.int32),
            pltpu.VMEM((B3B, AH), jnp.float32),
            pltpu.VMEM((B3B, AH), jnp.float32),
            pltpu.SemaphoreType.DMA,
            pltpu.SemaphoreType.DMA,
            pltpu.SemaphoreType.DMA,
            pltpu.SemaphoreType.DMA,
            pltpu.SemaphoreType.DMA,
            pltpu.SemaphoreType.DMA,
        ],
    )(_b1_body)
    return kfn(ps[0], pd[0], ps[1], pd[1], src, dst)


E8 = E // 8
BN2 = 2000

def _stage_b2(q20, q21, elem8, kt, kmat, a1lt, a2r):
    grid = (E8 // BN2,)

    def body(q0_ref, q1_ref, el_ref, kt_ref, k_ref,
             a1_ref, a2_ref, o0_ref, o1_ref):
        el128 = jnp.dot(el_ref[...], kt_ref[...],
                        preferred_element_type=jnp.float32)
        for h, (q_ref, o_ref) in enumerate(
                ((q0_ref, o0_ref), (q1_ref, o1_ref))):
            hid = jnp.maximum(q_ref[...] + el128 * a1_ref[h], 0.0)
            sc = (jnp.dot(hid, k_ref[h], preferred_element_type=jnp.float32)
                  + a2_ref[h, 0, 0])
            o_ref[...] = jnp.exp(jnp.maximum(sc, 0.2 * sc))

    return pl.pallas_call(
        body,
        grid=grid,
        in_specs=[
            pl.BlockSpec((BN2, 8 * AH), lambda i: (i, 0)),
            pl.BlockSpec((BN2, 8 * AH), lambda i: (i, 0)),
            pl.BlockSpec((BN2, 8), lambda i: (i, 0)),
            pl.BlockSpec((8, 8 * AH), lambda i: (0, 0)),
            pl.BlockSpec((H, 8 * AH, 8), lambda i: (0, 0, 0)),
            pl.BlockSpec((H, 1, 8 * AH), lambda i: (0, 0, 0)),
            pl.BlockSpec((H, 1, 1), lambda i: (0, 0, 0)),
        ],
        out_specs=[
            pl.BlockSpec((BN2, 8), lambda i: (i, 0)),
            pl.BlockSpec((BN2, 8), lambda i: (i, 0)),
        ],
        out_shape=[jax.ShapeDtypeStruct((E8, 8), jnp.float32)] * 2,
    )(q20, q21, elem8, kt, kmat, a1lt, a2r)


B3B = 128
NB3 = 78
EPW3 = NB3 * B3B
NT = NB3 // 2
EXT_BASE = NW * EPW3
NEXT = (E - EXT_BASE) // B3B


def _b3_body(feat0, feat1, ewr0, ewr1, srcA, dstA, zf, zr,
             pooled0, rsum0, pooled1, rsum1,
             srcb0, dstb0, ewrb0, fbuf0, rsb0,
             srcb1, dstb1, ewrb1, fbuf1, rsb1,
             pool_sp, rs_sp,
             lsem0, lsem1, gsem0, gsem1):
    c = lax.axis_index("c")
    s = lax.axis_index("s")
    wid = s * NC + c
    ebase = wid * EPW3
    rbase = s * RPT

    sets = ((srcb0, dstb0, ewrb0, fbuf0, rsb0, lsem0, gsem0),
            (srcb1, dstb1, ewrb1, fbuf1, rsb1, lsem1, gsem1))

    def issue_ld(b, p, ewr_h):
        srcb, dstb, ewrb, _, _, lsem, _ = sets[p]
        base = ebase + b * B3B
        pltpu.async_copy(srcA.at[pl.ds(base, B3B)], srcb, lsem)
        pltpu.async_copy(dstA.at[pl.ds(base, B3B)], dstb, lsem)
        pltpu.async_copy(ewr_h.at[pl.ds(base, B3B)], ewrb, lsem)

    def drain_ld(p, ewr_h):
        srcb, dstb, ewrb, _, _, lsem, _ = sets[p]
        pltpu.make_async_copy(srcA.at[pl.ds(0, B3B)], srcb, lsem).wait()
        pltpu.make_async_copy(dstA.at[pl.ds(0, B3B)], dstb, lsem).wait()
        pltpu.make_async_copy(ewr_h.at[pl.ds(0, B3B)], ewrb, lsem).wait()

    def issue_gather(p, feat_h):
        _, dstb, _, fbuf, _, _, gsem = sets[p]
        pltpu.async_copy(feat_h.at[dstb], fbuf, gsem)

    def drain_gather(p):
        _, _, _, fbuf, _, _, gsem = sets[p]
        pltpu.make_async_copy(zf.at[pl.ds(0, B3B)], fbuf, gsem).wait()

    def scale_scatter(p):
        srcb, _, ewrb, fbuf, rsb, _, _ = sets[p]
        def scl(g, cy):
            ewv = ewrb[pl.ds(g * 16, 16)]
            for l in range(16):
                e = g * 16 + l
                wv = jnp.full((AH,), ewv[l])
                rsb[e, :] = wv
                for j in range(D // AH):
                    fbuf[e, pl.ds(j * AH, AH)] = (
                        fbuf[e, pl.ds(j * AH, AH)] * wv)
            return cy
        lax.fori_loop(0, B3B // 16, scl, 0)
        pltpu.sync_copy(fbuf, pool_sp.at[srcb], add=True)
        pltpu.sync_copy(rsb, rs_sp.at[srcb], add=True)

    for feat_h, ewr_h, pooled_h, rsum_h in (
            (feat0, ewr0, pooled0, rsum0),
            (feat1, ewr1, pooled1, rsum1)):

        for k in range(RPT // B3B):
            pltpu.sync_copy(zf.at[pl.ds(rbase + k * B3B, B3B)],
                            pool_sp.at[pl.ds(rbase + k * B3B, B3B)])
            pltpu.sync_copy(zr.at[pl.ds(rbase + k * B3B, B3B)],
                            rs_sp.at[pl.ds(rbase + k * B3B, B3B)])
        plsc.subcore_barrier()

        issue_ld(0, 0, ewr_h)
        drain_ld(0, ewr_h)
        issue_gather(0, feat_h)
        issue_ld(1, 1, ewr_h)

        def it(t, carry, feat_h=feat_h, ewr_h=ewr_h):
            drain_ld(1, ewr_h)
            issue_gather(1, feat_h)
            drain_gather(0)
            scale_scatter(0)
            @pl.when(t < NT - 1)
            def _():
                issue_ld(2 * (t + 1), 0, ewr_h)
            @pl.when(t < NT - 1)
            def _():
                drain_ld(0, ewr_h)
                issue_gather(0, feat_h)
            drain_gather(1)
            scale_scatter(1)
            @pl.when(t < NT - 1)
            def _():
                issue_ld(2 * (t + 1) + 1, 1, ewr_h)
            return carry
        lax.fori_loop(0, NT, it, 0)

        @pl.when(wid < NEXT)
        def _(feat_h=feat_h, ewr_h=ewr_h):
            tb = EXT_BASE + wid * B3B
            pltpu.sync_copy(srcA.at[pl.ds(tb, B3B)], srcb0)
            pltpu.sync_copy(dstA.at[pl.ds(tb, B3B)], dstb0)
            pltpu.sync_copy(ewr_h.at[pl.ds(tb, B3B)], ewrb0)
            pltpu.async_copy(feat_h.at[dstb0], fbuf0, gsem0).wait()
            scale_scatter(0)

        plsc.subcore_barrier()
        pltpu.sync_copy(pool_sp.at[pl.ds(rbase, RPT)],
                        pooled_h.at[c, pl.ds(rbase, RPT)])
        pltpu.sync_copy(rs_sp.at[pl.ds(rbase, RPT)],
                        rsum_h.at[c, pl.ds(rbase, RPT)])
        plsc.subcore_barrier()


def _stage_b3(feat, ewr0, ewr1, src, dst, zf, zr):
    mesh = plsc.VectorSubcoreMesh(core_axis_name="c", subcore_axis_name="s")
    kfn = functools.partial(
        pl.kernel,
        mesh=mesh,
        compiler_params=pltpu.CompilerParams(use_tc_tiling_on_sc=False),
        out_type=[
            jax.ShapeDtypeStruct((NC, N2, D), jnp.float32),
            jax.ShapeDtypeStruct((NC, N2, AH), jnp.float32),
            jax.ShapeDtypeStruct((NC, N2, D), jnp.float32),
            jax.ShapeDtypeStruct((NC, N2, AH), jnp.float32),
        ],
        scratch_types=[
            pltpu.VMEM((B3B,), jnp.int32),
            pltpu.VMEM((B3B,), jnp.int32),
            pltpu.VMEM((B3B,), jnp.float32),
            pltpu.VMEM((B3B, D), jnp.float32),
            pltpu.VMEM((B3B, AH), jnp.float32),
            pltpu.VMEM((B3B,), jnp.int32),
            pltpu.VMEM((B3B,), jnp.int32),
            pltpu.VMEM((B3B,), jnp.float32),
            pltpu.VMEM((B3B, D), jnp.float32),
            pltpu.VMEM((B3B, AH), jnp.float32),
            pltpu.VMEM_SHARED((N2, D), jnp.float32),
            pltpu.VMEM_SHARED((N2, AH), jnp.float32),
            pltpu.SemaphoreType.DMA,
            pltpu.SemaphoreType.DMA,
            pltpu.SemaphoreType.DMA,
            pltpu.SemaphoreType.DMA,
        ],
    )(_b3_body)
    return kfn(feat[0], feat[1], ewr0, ewr1, src, dst, zf, zr)


def _stage_c(p0, r0, p1, r1):
    BN = 1000
    grid = (N // BN,)

    def body(p0_ref, r0_ref, p1_ref, r1_ref, out_ref):
        for h, (p, r) in enumerate(((p0_ref, r0_ref), (p1_ref, r1_ref))):
            pooled = p[0] + p[1]
            rs = r[0, :, 0] + r[1, :, 0] + 1e-10
            out_ref[:, h * D:(h + 1) * D] = pooled / rs[:, None]

    return pl.pallas_call(
        body,
        grid=grid,
        in_specs=[
            pl.BlockSpec((NC, BN, D), lambda i: (0, i, 0)),
            pl.BlockSpec((NC, BN, AH), lambda i: (0, i, 0)),
            pl.BlockSpec((NC, BN, D), lambda i: (0, i, 0)),
            pl.BlockSpec((NC, BN, AH), lambda i: (0, i, 0)),
        ],
        out_specs=pl.BlockSpec((BN, H * D), lambda i: (i, 0)),
        out_shape=jax.ShapeDtypeStruct((N, H * D), jnp.float32),
    )(p0, r0, p1, r1)


def kernel(x, idx, elem, W1, b1, W2, b2, A1, a1, A2, a2):
    A1a = A1[:, :D, :]
    A1b = A1[:, D:2 * D, :]
    feat, ps, pd = _stage_a(x, W1, b1, W2, b2, A1a, a1, A1b)
    src = idx[0]
    dst = idx[1]
    qr0, qr1 = _stage_b1(ps, pd, src, dst)
    kt = jnp.kron(jnp.eye(8, dtype=jnp.float32),
                  jnp.ones((1, AH), jnp.float32))
    a2vt = jnp.tile(A2[:, :, 0], (1, 8))
    kmat = kt.T[None, :, :] * a2vt[:, :, None]
    a1lt = jnp.tile(A1[:, 2 * D, :], (1, 8))[:, None, :]
    a2r = a2[:, :, None]
    ew20, ew21 = _stage_b2(qr0.reshape(E8, 8 * AH), qr1.reshape(E8, 8 * AH),
                           elem.reshape(E8, 8), kt, kmat, a1lt, a2r)
    zf = jnp.zeros((N2, D), jnp.float32)
    zr = jnp.zeros((N2, AH), jnp.float32)
    p0, r0, p1, r1 = _stage_b3(feat, ew20.reshape(E), ew21.reshape(E),
                               src, dst, zf, zr)
    return _stage_c(p0, r0, p1, r1)


# speedup vs baseline: 1.1086x; 1.1086x over previous
import functools

import jax
import jax.numpy as jnp
from jax import lax
from jax.experimental import pallas as pl
from jax.experimental.pallas import tpu as pltpu
from jax.experimental.pallas import tpu_sc as plsc

N = 10000
E = 320000
D = 128
H = 2
AH = 16

NC = 2
NS = 16
NW = NC * NS
EPW = E // NW
B = 80
NB = EPW // B
N2 = 10240
RPT = N2 // NS
BE = 8000


def _stage_a(x, W1, b1, W2, b2, A1a, a1v, A1b):
    BN = 1000
    grid = (H, N // BN)

    def body(x_ref, w1_ref, b1_ref, w2_ref, b2_ref, a1a_ref, a1_ref,
             a1b_ref, feat_ref, ps_ref, pd_ref):
        xb = x_ref[...]
        f1 = jnp.maximum(
            jnp.dot(xb, w1_ref[0], preferred_element_type=jnp.float32)
            + b1_ref[0], 0.0)
        ft = (jnp.dot(f1, w2_ref[0], preferred_element_type=jnp.float32)
              + b2_ref[0])
        feat_ref[0] = ft
        ps_ref[0] = jnp.dot(ft, a1a_ref[0], preferred_element_type=jnp.float32)
        pd_ref[0] = (jnp.dot(ft, a1b_ref[0], preferred_element_type=jnp.float32)
                     + a1_ref[0])

    return pl.pallas_call(
        body,
        grid=grid,
        in_specs=[
            pl.BlockSpec((BN, D), lambda h, i: (i, 0)),
            pl.BlockSpec((1, D, D), lambda h, i: (h, 0, 0)),
            pl.BlockSpec((1, 1, D), lambda h, i: (h, 0, 0)),
            pl.BlockSpec((1, D, D), lambda h, i: (h, 0, 0)),
            pl.BlockSpec((1, 1, D), lambda h, i: (h, 0, 0)),
            pl.BlockSpec((1, D, AH), lambda h, i: (h, 0, 0)),
            pl.BlockSpec((1, 1, AH), lambda h, i: (h, 0, 0)),
            pl.BlockSpec((1, D, AH), lambda h, i: (h, 0, 0)),
        ],
        out_specs=[
            pl.BlockSpec((1, BN, D), lambda h, i: (h, i, 0)),
            pl.BlockSpec((1, BN, AH), lambda h, i: (h, i, 0)),
            pl.BlockSpec((1, BN, AH), lambda h, i: (h, i, 0)),
        ],
        out_shape=[
            jax.ShapeDtypeStruct((H, N, D), jnp.float32),
            jax.ShapeDtypeStruct((H, N, AH), jnp.float32),
            jax.ShapeDtypeStruct((H, N, AH), jnp.float32),
        ],
    )(x, W1, b1[:, None, :], W2, b2[:, None, :], A1a, a1v[:, None, :], A1b)


def _b1_body(ps0, pd0, ps1, pd1, srcA, dstA,
             qr0, qr1,
             srcb0, dstb0, g10, g20,
             srcb1, dstb1, g11, g21,
             lsem0, lsem1, gsem0, gsem1, wsem0, wsem1):
    c = lax.axis_index("c")
    s = lax.axis_index("s")
    wid = s * NC + c
    ebase = wid * EPW3

    sets = ((srcb0, dstb0, g10, g20, lsem0, gsem0, wsem0),
            (srcb1, dstb1, g11, g21, lsem1, gsem1, wsem1))

    def issue_ld(b, p):
        srcb, dstb, _, _, lsem, _, _ = sets[p]
        base = ebase + b * B3B
        pltpu.async_copy(srcA.at[pl.ds(base, B3B)], srcb, lsem)
        pltpu.async_copy(dstA.at[pl.ds(base, B3B)], dstb, lsem)

    def drain_ld(p):
        srcb, dstb, _, _, lsem, _, _ = sets[p]
        pltpu.make_async_copy(srcA.at[pl.ds(0, B3B)], srcb, lsem).wait()
        pltpu.make_async_copy(dstA.at[pl.ds(0, B3B)], dstb, lsem).wait()

    def issue_gather(p, ps_h, pd_h):
        srcb, dstb, g1, g2, _, gsem, _ = sets[p]
        pltpu.async_copy(ps_h.at[srcb], g1, gsem)
        pltpu.async_copy(pd_h.at[dstb], g2, gsem)

    def drain_gather(p, ps_h):
        _, _, g1, g2, _, gsem, _ = sets[p]
        pltpu.make_async_copy(ps_h.at[pl.ds(0, B3B)], g1, gsem).wait()
        pltpu.make_async_copy(ps_h.at[pl.ds(0, B3B)], g2, gsem).wait()

    def drain_write(p, qr_h):
        _, _, g1, _, _, _, wsem = sets[p]
        pltpu.make_async_copy(g1, qr_h.at[pl.ds(0, B3B)], wsem).wait()

    def add_write(b, p, qr_h):
        _, _, g1, g2, _, _, wsem = sets[p]
        def add_e(e, cy):
            g1[e, :] = g1[e, :] + g2[e, :]
            return cy
        lax.fori_loop(0, B3B, add_e, 0)
        pltpu.async_copy(g1, qr_h.at[pl.ds(ebase + b * B3B, B3B)], wsem)

    for ps_h, pd_h, qr_h in ((ps0, pd0, qr0), (ps1, pd1, qr1)):
        issue_ld(0, 0)
        drain_ld(0)
        issue_gather(0, ps_h, pd_h)
        issue_ld(1, 1)

        def it(t, carry, ps_h=ps_h, pd_h=pd_h, qr_h=qr_h):
            drain_ld(1)
            @pl.when(t > 0)
            def _():
                drain_write(1, qr_h)
            issue_gather(1, ps_h, pd_h)
            drain_gather(0, ps_h)
            add_write(2 * t, 0, qr_h)
            @pl.when(t < NT - 1)
            def _():
                issue_ld(2 * (t + 1), 0)
            @pl.when(t < NT - 1)
            def _():
                drain_ld(0)
                drain_write(0, qr_h)
                issue_gather(0, ps_h, pd_h)
            drain_gather(1, ps_h)
            add_write(2 * t + 1, 1, qr_h)
            @pl.when(t < NT - 1)
            def _():
                issue_ld(2 * (t + 1) + 1, 1)
            return carry
        lax.fori_loop(0, NT, it, 0)
        drain_write(0, qr_h)
        drain_write(1, qr_h)

        @pl.when(wid < NEXT)
        def _(ps_h=ps_h, pd_h=pd_h, qr_h=qr_h):
            tb = EXT_BASE + wid * B3B
            pltpu.sync_copy(srcA.at[pl.ds(tb, B3B)], srcb0)
            pltpu.sync_copy(dstA.at[pl.ds(tb, B3B)], dstb0)
            cp1 = pltpu.async_copy(ps_h.at[srcb0], g10, gsem0)
            cp2 = pltpu.async_copy(pd_h.at[dstb0], g20, gsem0)
            cp1.wait()
            cp2.wait()
            def add_e(e, cy):
                g10[e, :] = g10[e, :] + g20[e, :]
                return cy
            lax.fori_loop(0, B3B, add_e, 0)
            pltpu.sync_copy(g10, qr_h.at[pl.ds(tb, B3B)])


def _stage_b1(ps, pd, src, dst):
    mesh = plsc.VectorSubcoreMesh(core_axis_name="c", subcore_axis_name="s")
    kfn = functools.partial(
        pl.kernel,
        mesh=mesh,
        compiler_params=pltpu.CompilerParams(use_tc_tiling_on_sc=False),
        out_type=[jax.ShapeDtypeStruct((E, AH), jnp.float32)] * 2,
        scratch_types=[
            pltpu.VMEM((B3B,), jnp.int32),
            pltpu.VMEM((B3B,), jnp.int32),
            pltpu.VMEM((B3B, AH), jnp.float32),
            pltpu.VMEM((B3B, AH), jnp.float32),
            pltpu.VMEM((B3B,), jnp.int32),
            pltpu.VMEM((B3B,), jnp.int32),
            pltpu.VMEM((B3B, AH), jnp.float32),
            pltpu.VMEM((B3B, AH), jnp.float32),
            pltpu.SemaphoreType.DMA,
            pltpu.SemaphoreType.DMA,
            pltpu.SemaphoreType.DMA,
            pltpu.SemaphoreType.DMA,
            pltpu.SemaphoreType.DMA,
            pltpu.SemaphoreType.DMA,
        ],
    )(_b1_body)
    return kfn(ps[0], pd[0], ps[1], pd[1], src, dst)


E8 = E // 8
BN2 = 2000

def _stage_b2(q20, q21, elem8, kt, kmat, a1lt, a2r):
    grid = (E8 // BN2,)

    def body(q0_ref, q1_ref, el_ref, kt_ref, k_ref,
             a1_ref, a2_ref, o0_ref, o1_ref):
        el128 = jnp.dot(el_ref[...], kt_ref[...],
                        preferred_element_type=jnp.float32)
        for h, (q_ref, o_ref) in enumerate(
                ((q0_ref, o0_ref), (q1_ref, o1_ref))):
            hid = jnp.maximum(q_ref[...] + el128 * a1_ref[h], 0.0)
            sc = (jnp.dot(hid, k_ref[h], preferred_element_type=jnp.float32)
                  + a2_ref[h, 0, 0])
            o_ref[...] = jnp.exp(jnp.maximum(sc, 0.2 * sc))

    return pl.pallas_call(
        body,
        grid=grid,
        in_specs=[
            pl.BlockSpec((BN2, 8 * AH), lambda i: (i, 0)),
            pl.BlockSpec((BN2, 8 * AH), lambda i: (i, 0)),
            pl.BlockSpec((BN2, 8), lambda i: (i, 0)),
            pl.BlockSpec((8, 8 * AH), lambda i: (0, 0)),
            pl.BlockSpec((H, 8 * AH, 8), lambda i: (0, 0, 0)),
            pl.BlockSpec((H, 1, 8 * AH), lambda i: (0, 0, 0)),
            pl.BlockSpec((H, 1, 1), lambda i: (0, 0, 0)),
        ],
        out_specs=[
            pl.BlockSpec((BN2, 8), lambda i: (i, 0)),
            pl.BlockSpec((BN2, 8), lambda i: (i, 0)),
        ],
        out_shape=[jax.ShapeDtypeStruct((E8, 8), jnp.float32)] * 2,
    )(q20, q21, elem8, kt, kmat, a1lt, a2r)


B3B = 128
NB3 = 78
EPW3 = NB3 * B3B
NT = NB3 // 2
EXT_BASE = NW * EPW3
NEXT = (E - EXT_BASE) // B3B


NT6 = NB3 // 6


def _b3_body(feat0, feat1, ewr0, ewr1, srcA, dstA, zf, zr,
             pooled0, rsum0, pooled1, rsum1,
             srcb0, dstb0, ewrb0, srcb1, dstb1, ewrb1,
             srcb2, dstb2, ewrb2,
             fbuf0, rsb0, fbuf1, rsb1,
             pool_sp, rs_sp,
             lsem0, lsem1, lsem2, gsem0, gsem1, ssem0, ssem1):
    c = lax.axis_index("c")
    s = lax.axis_index("s")
    wid = s * NC + c
    ebase = wid * EPW3
    rbase = s * RPT

    isets = ((srcb0, dstb0, ewrb0, lsem0),
             (srcb1, dstb1, ewrb1, lsem1),
             (srcb2, dstb2, ewrb2, lsem2))
    dsets = ((fbuf0, rsb0, gsem0, ssem0),
             (fbuf1, rsb1, gsem1, ssem1))

    def issue_ld(b, i, ewr_h):
        srcb, dstb, ewrb, lsem = isets[i]
        base = ebase + b * B3B
        pltpu.async_copy(srcA.at[pl.ds(base, B3B)], srcb, lsem)
        pltpu.async_copy(dstA.at[pl.ds(base, B3B)], dstb, lsem)
        pltpu.async_copy(ewr_h.at[pl.ds(base, B3B)], ewrb, lsem)

    def drain_ld(i, ewr_h):
        srcb, dstb, ewrb, lsem = isets[i]
        pltpu.make_async_copy(srcA.at[pl.ds(0, B3B)], srcb, lsem).wait()
        pltpu.make_async_copy(dstA.at[pl.ds(0, B3B)], dstb, lsem).wait()
        pltpu.make_async_copy(ewr_h.at[pl.ds(0, B3B)], ewrb, lsem).wait()

    def issue_gather(i, d, feat_h):
        fbuf, _, gsem, _ = dsets[d]
        pltpu.async_copy(feat_h.at[isets[i][1]], fbuf, gsem)

    def drain_gather(d):
        fbuf, _, gsem, _ = dsets[d]
        pltpu.make_async_copy(zf.at[pl.ds(0, B3B)], fbuf, gsem).wait()

    def scale(i, d):
        ewrb = isets[i][2]
        fbuf, rsb, _, _ = dsets[d]
        def scl(g, cy):
            ewv = ewrb[pl.ds(g * 16, 16)]
            for l in range(16):
                e = g * 16 + l
                wv = jnp.full((AH,), ewv[l])
                rsb[e, :] = wv
                for j in range(D // AH):
                    fbuf[e, pl.ds(j * AH, AH)] = (
                        fbuf[e, pl.ds(j * AH, AH)] * wv)
            return cy
        lax.fori_loop(0, B3B // 16, scl, 0)

    def issue_scatter(i, d):
        srcb = isets[i][0]
        fbuf, rsb, _, ssem = dsets[d]
        pltpu.async_copy(fbuf, pool_sp.at[srcb], ssem, add=True)
        pltpu.async_copy(rsb, rs_sp.at[srcb], ssem, add=True)

    def drain_scatter(d):
        fbuf, rsb, _, ssem = dsets[d]
        pltpu.make_async_copy(fbuf, pool_sp.at[pl.ds(0, B3B)], ssem).wait()
        pltpu.make_async_copy(rsb, rs_sp.at[pl.ds(0, B3B)], ssem).wait()

    for feat_h, ewr_h, pooled_h, rsum_h in (
            (feat0, ewr0, pooled0, rsum0),
            (feat1, ewr1, pooled1, rsum1)):

        for k in range(RPT // B3B):
            pltpu.sync_copy(zf.at[pl.ds(rbase + k * B3B, B3B)],
                            pool_sp.at[pl.ds(rbase + k * B3B, B3B)])
            pltpu.sync_copy(zr.at[pl.ds(rbase + k * B3B, B3B)],
                            rs_sp.at[pl.ds(rbase + k * B3B, B3B)])
        plsc.subcore_barrier()

        issue_ld(0, 0, ewr_h)
        drain_ld(0, ewr_h)
        issue_gather(0, 0, feat_h)
        issue_ld(1, 1, ewr_h)

        def it(t, carry, feat_h=feat_h, ewr_h=ewr_h):
            for ph in range(6):
                k = 6 * t + ph
                i, i1, i2 = ph % 3, (ph + 1) % 3, (ph + 2) % 3
                d, d1 = ph % 2, (ph + 1) % 2
                if ph == 5:
                    @pl.when(t < NT6 - 1)
                    def _():
                        drain_ld(i1, ewr_h)
                else:
                    drain_ld(i1, ewr_h)
                if ph == 0:
                    @pl.when(t > 0)
                    def _():
                        drain_scatter(d1)
                else:
                    drain_scatter(d1)
                if ph == 5:
                    @pl.when(t < NT6 - 1)
                    def _():
                        issue_gather(i1, d1, feat_h)
                else:
                    issue_gather(i1, d1, feat_h)
                if ph >= 4:
                    @pl.when(t < NT6 - 1)
                    def _():
                        issue_ld(k + 2, i2, ewr_h)
                else:
                    issue_ld(k + 2, i2, ewr_h)
                drain_gather(d)
                scale(i, d)
                issue_scatter(i, d)
            return carry
        lax.fori_loop(0, NT6, it, 0)
        drain_scatter(1)

        @pl.when(wid < NEXT)
        def _(feat_h=feat_h, ewr_h=ewr_h):
            tb = EXT_BASE + wid * B3B
            pltpu.sync_copy(srcA.at[pl.ds(tb, B3B)], srcb0)
            pltpu.sync_copy(dstA.at[pl.ds(tb, B3B)], dstb0)
            pltpu.sync_copy(ewr_h.at[pl.ds(tb, B3B)], ewrb0)
            pltpu.async_copy(feat_h.at[dstb0], fbuf0, gsem0).wait()
            scale(0, 0)
            pltpu.sync_copy(fbuf0, pool_sp.at[srcb0], add=True)
            pltpu.sync_copy(rsb0, rs_sp.at[srcb0], add=True)

        plsc.subcore_barrier()
        pltpu.sync_copy(pool_sp.at[pl.ds(rbase, RPT)],
                        pooled_h.at[c, pl.ds(rbase, RPT)])
        pltpu.sync_copy(rs_sp.at[pl.ds(rbase, RPT)],
                        rsum_h.at[c, pl.ds(rbase, RPT)])
        plsc.subcore_barrier()


def _stage_b3(feat, ewr0, ewr1, src, dst, zf, zr):
    mesh = plsc.VectorSubcoreMesh(core_axis_name="c", subcore_axis_name="s")
    kfn = functools.partial(
        pl.kernel,
        mesh=mesh,
        compiler_params=pltpu.CompilerParams(use_tc_tiling_on_sc=False),
        out_type=[
            jax.ShapeDtypeStruct((NC, N2, D), jnp.float32),
            jax.ShapeDtypeStruct((NC, N2, AH), jnp.float32),
            jax.ShapeDtypeStruct((NC, N2, D), jnp.float32),
            jax.ShapeDtypeStruct((NC, N2, AH), jnp.float32),
        ],
        scratch_types=[
            pltpu.VMEM((B3B,), jnp.int32),
            pltpu.VMEM((B3B,), jnp.int32),
            pltpu.VMEM((B3B,), jnp.float32),
            pltpu.VMEM((B3B,), jnp.int32),
            pltpu.VMEM((B3B,), jnp.int32),
            pltpu.VMEM((B3B,), jnp.float32),
            pltpu.VMEM((B3B,), jnp.int32),
            pltpu.VMEM((B3B,), jnp.int32),
            pltpu.VMEM((B3B,), jnp.float32),
            pltpu.VMEM((B3B, D), jnp.float32),
            pltpu.VMEM((B3B, AH), jnp.float32),
            pltpu.VMEM((B3B, D), jnp.float32),
            pltpu.VMEM((B3B, AH), jnp.float32),
            pltpu.VMEM_SHARED((N2, D), jnp.float32),
            pltpu.VMEM_SHARED((N2, AH), jnp.float32),
            pltpu.SemaphoreType.DMA,
            pltpu.SemaphoreType.DMA,
            pltpu.SemaphoreType.DMA,
            pltpu.SemaphoreType.DMA,
            pltpu.SemaphoreType.DMA,
            pltpu.SemaphoreType.DMA,
            pltpu.SemaphoreType.DMA,
        ],
    )(_b3_body)
    return kfn(feat[0], feat[1], ewr0, ewr1, src, dst, zf, zr)


def _stage_c(p0, r0, p1, r1):
    BN = 1000
    grid = (N // BN,)

    def body(p0_ref, r0_ref, p1_ref, r1_ref, out_ref):
        for h, (p, r) in enumerate(((p0_ref, r0_ref), (p1_ref, r1_ref))):
            pooled = p[0] + p[1]
            rs = r[0, :, 0] + r[1, :, 0] + 1e-10
            out_ref[:, h * D:(h + 1) * D] = pooled / rs[:, None]

    return pl.pallas_call(
        body,
        grid=grid,
        in_specs=[
            pl.BlockSpec((NC, BN, D), lambda i: (0, i, 0)),
            pl.BlockSpec((NC, BN, AH), lambda i: (0, i, 0)),
            pl.BlockSpec((NC, BN, D), lambda i: (0, i, 0)),
            pl.BlockSpec((NC, BN, AH), lambda i: (0, i, 0)),
        ],
        out_specs=pl.BlockSpec((BN, H * D), lambda i: (i, 0)),
        out_shape=jax.ShapeDtypeStruct((N, H * D), jnp.float32),
    )(p0, r0, p1, r1)


def kernel(x, idx, elem, W1, b1, W2, b2, A1, a1, A2, a2):
    A1a = A1[:, :D, :]
    A1b = A1[:, D:2 * D, :]
    feat, ps, pd = _stage_a(x, W1, b1, W2, b2, A1a, a1, A1b)
    src = idx[0]
    dst = idx[1]
    qr0, qr1 = _stage_b1(ps, pd, src, dst)
    kt = jnp.kron(jnp.eye(8, dtype=jnp.float32),
                  jnp.ones((1, AH), jnp.float32))
    a2vt = jnp.tile(A2[:, :, 0], (1, 8))
    kmat = kt.T[None, :, :] * a2vt[:, :, None]
    a1lt = jnp.tile(A1[:, 2 * D, :], (1, 8))[:, None, :]
    a2r = a2[:, :, None]
    ew20, ew21 = _stage_b2(qr0.reshape(E8, 8 * AH), qr1.reshape(E8, 8 * AH),
                           elem.reshape(E8, 8), kt, kmat, a1lt, a2r)
    zf = jnp.zeros((N2, D), jnp.float32)
    zr = jnp.zeros((N2, AH), jnp.float32)
    p0, r0, p1, r1 = _stage_b3(feat, ew20.reshape(E), ew21.reshape(E),
                               src, dst, zf, zr)
    return _stage_c(p0, r0, p1, r1)


# speedup vs baseline: 1.1639x; 1.0498x over previous
import functools

import jax
import jax.numpy as jnp
from jax import lax
from jax.experimental import pallas as pl
from jax.experimental.pallas import tpu as pltpu
from jax.experimental.pallas import tpu_sc as plsc

N = 10000
E = 320000
D = 128
H = 2
AH = 16

NC = 2
NS = 16
NW = NC * NS
EPW = E // NW
B = 80
NB = EPW // B
N2 = 10240
RPT = N2 // NS
BE = 8000


def _stage_a(x, W1, b1, W2, b2, A1a, a1v, A1b):
    BN = 1000
    grid = (H, N // BN)

    def body(x_ref, w1_ref, b1_ref, w2_ref, b2_ref, a1a_ref, a1_ref,
             a1b_ref, feat_ref, ps_ref, pd_ref):
        xb = x_ref[...]
        f1 = jnp.maximum(
            jnp.dot(xb, w1_ref[0], preferred_element_type=jnp.float32)
            + b1_ref[0], 0.0)
        ft = (jnp.dot(f1, w2_ref[0], preferred_element_type=jnp.float32)
              + b2_ref[0])
        feat_ref[0] = ft
        ps_ref[0] = jnp.dot(ft, a1a_ref[0], preferred_element_type=jnp.float32)
        pd_ref[0] = (jnp.dot(ft, a1b_ref[0], preferred_element_type=jnp.float32)
                     + a1_ref[0])

    return pl.pallas_call(
        body,
        grid=grid,
        in_specs=[
            pl.BlockSpec((BN, D), lambda h, i: (i, 0)),
            pl.BlockSpec((1, D, D), lambda h, i: (h, 0, 0)),
            pl.BlockSpec((1, 1, D), lambda h, i: (h, 0, 0)),
            pl.BlockSpec((1, D, D), lambda h, i: (h, 0, 0)),
            pl.BlockSpec((1, 1, D), lambda h, i: (h, 0, 0)),
            pl.BlockSpec((1, D, AH), lambda h, i: (h, 0, 0)),
            pl.BlockSpec((1, 1, AH), lambda h, i: (h, 0, 0)),
            pl.BlockSpec((1, D, AH), lambda h, i: (h, 0, 0)),
        ],
        out_specs=[
            pl.BlockSpec((1, BN, D), lambda h, i: (h, i, 0)),
            pl.BlockSpec((1, BN, AH), lambda h, i: (h, i, 0)),
            pl.BlockSpec((1, BN, AH), lambda h, i: (h, i, 0)),
        ],
        out_shape=[
            jax.ShapeDtypeStruct((H, N, D), jnp.float32),
            jax.ShapeDtypeStruct((H, N, AH), jnp.float32),
            jax.ShapeDtypeStruct((H, N, AH), jnp.float32),
        ],
    )(x, W1, b1[:, None, :], W2, b2[:, None, :], A1a, a1v[:, None, :], A1b)


def _b1_body(ps0, pd0, ps1, pd1, srcA, dstA,
             qr0, qr1,
             srcb0, dstb0, g10, g20,
             srcb1, dstb1, g11, g21,
             lsem0, lsem1, gsem0, gsem1, wsem0, wsem1):
    c = lax.axis_index("c")
    s = lax.axis_index("s")
    wid = s * NC + c
    ebase = wid * EPW3

    sets = ((srcb0, dstb0, g10, g20, lsem0, gsem0, wsem0),
            (srcb1, dstb1, g11, g21, lsem1, gsem1, wsem1))

    def issue_ld(b, p):
        srcb, dstb, _, _, lsem, _, _ = sets[p]
        base = ebase + b * B3B
        pltpu.async_copy(srcA.at[pl.ds(base, B3B)], srcb, lsem)
        pltpu.async_copy(dstA.at[pl.ds(base, B3B)], dstb, lsem)

    def drain_ld(p):
        srcb, dstb, _, _, lsem, _, _ = sets[p]
        pltpu.make_async_copy(srcA.at[pl.ds(0, B3B)], srcb, lsem).wait()
        pltpu.make_async_copy(dstA.at[pl.ds(0, B3B)], dstb, lsem).wait()

    def issue_gather(p, ps_h, pd_h):
        srcb, dstb, g1, g2, _, gsem, _ = sets[p]
        pltpu.async_copy(ps_h.at[srcb], g1, gsem)
        pltpu.async_copy(pd_h.at[dstb], g2, gsem)

    def drain_gather(p, ps_h):
        _, _, g1, g2, _, gsem, _ = sets[p]
        pltpu.make_async_copy(ps_h.at[pl.ds(0, B3B)], g1, gsem).wait()
        pltpu.make_async_copy(ps_h.at[pl.ds(0, B3B)], g2, gsem).wait()

    def drain_write(p, qr_h):
        _, _, g1, _, _, _, wsem = sets[p]
        pltpu.make_async_copy(g1, qr_h.at[pl.ds(0, B3B)], wsem).wait()

    def add_write(b, p, qr_h):
        _, _, g1, g2, _, _, wsem = sets[p]
        def add_e(g, cy):
            for l in range(4):
                e = g * 4 + l
                g1[e, :] = g1[e, :] + g2[e, :]
            return cy
        lax.fori_loop(0, B3B // 4, add_e, 0)
        pltpu.async_copy(g1, qr_h.at[pl.ds(ebase + b * B3B, B3B)], wsem)

    for ps_h, pd_h, qr_h in ((ps0, pd0, qr0), (ps1, pd1, qr1)):
        issue_ld(0, 0)
        drain_ld(0)
        issue_gather(0, ps_h, pd_h)
        issue_ld(1, 1)

        def it(t, carry, ps_h=ps_h, pd_h=pd_h, qr_h=qr_h):
            drain_ld(1)
            @pl.when(t > 0)
            def _():
                drain_write(1, qr_h)
            issue_gather(1, ps_h, pd_h)
            drain_gather(0, ps_h)
            add_write(2 * t, 0, qr_h)
            @pl.when(t < NT - 1)
            def _():
                issue_ld(2 * (t + 1), 0)
            @pl.when(t < NT - 1)
            def _():
                drain_ld(0)
                drain_write(0, qr_h)
                issue_gather(0, ps_h, pd_h)
            drain_gather(1, ps_h)
            add_write(2 * t + 1, 1, qr_h)
            @pl.when(t < NT - 1)
            def _():
                issue_ld(2 * (t + 1) + 1, 1)
            return carry
        lax.fori_loop(0, NT, it, 0)
        drain_write(0, qr_h)
        drain_write(1, qr_h)

        @pl.when(wid < NEXT)
        def _(ps_h=ps_h, pd_h=pd_h, qr_h=qr_h):
            tb = EXT_BASE + wid * B3B
            pltpu.sync_copy(srcA.at[pl.ds(tb, B3B)], srcb0)
            pltpu.sync_copy(dstA.at[pl.ds(tb, B3B)], dstb0)
            cp1 = pltpu.async_copy(ps_h.at[srcb0], g10, gsem0)
            cp2 = pltpu.async_copy(pd_h.at[dstb0], g20, gsem0)
            cp1.wait()
            cp2.wait()
            def add_e(g, cy):
                for l in range(4):
                    e = g * 4 + l
                    g10[e, :] = g10[e, :] + g20[e, :]
                return cy
            lax.fori_loop(0, B3B // 4, add_e, 0)
            pltpu.sync_copy(g10, qr_h.at[pl.ds(tb, B3B)])


def _stage_b1(ps, pd, src, dst):
    mesh = plsc.VectorSubcoreMesh(core_axis_name="c", subcore_axis_name="s")
    kfn = functools.partial(
        pl.kernel,
        mesh=mesh,
        compiler_params=pltpu.CompilerParams(use_tc_tiling_on_sc=False),
        out_type=[jax.ShapeDtypeStruct((E, AH), jnp.float32)] * 2,
        scratch_types=[
            pltpu.VMEM((B3B,), jnp.int32),
            pltpu.VMEM((B3B,), jnp.int32),
            pltpu.VMEM((B3B, AH), jnp.float32),
            pltpu.VMEM((B3B, AH), jnp.float32),
            pltpu.VMEM((B3B,), jnp.int32),
            pltpu.VMEM((B3B,), jnp.int32),
            pltpu.VMEM((B3B, AH), jnp.float32),
            pltpu.VMEM((B3B, AH), jnp.float32),
            pltpu.SemaphoreType.DMA,
            pltpu.SemaphoreType.DMA,
            pltpu.SemaphoreType.DMA,
            pltpu.SemaphoreType.DMA,
            pltpu.SemaphoreType.DMA,
            pltpu.SemaphoreType.DMA,
        ],
    )(_b1_body)
    return kfn(ps[0], pd[0], ps[1], pd[1], src, dst)


E8 = E // 8
BN2 = 2000

def _stage_b2(q20, q21, elem8, kt, kmat, a1lt, a2r):
    grid = (E8 // BN2,)

    def body(q0_ref, q1_ref, el_ref, kt_ref, k_ref,
             a1_ref, a2_ref, o0_ref, o1_ref):
        el128 = jnp.dot(el_ref[...], kt_ref[...],
                        preferred_element_type=jnp.float32)
        for h, (q_ref, o_ref) in enumerate(
                ((q0_ref, o0_ref), (q1_ref, o1_ref))):
            hid = jnp.maximum(q_ref[...] + el128 * a1_ref[h], 0.0)
            sc = (jnp.dot(hid, k_ref[h], preferred_element_type=jnp.float32)
                  + a2_ref[h, 0, 0])
            o_ref[...] = jnp.exp(jnp.maximum(sc, 0.2 * sc))

    return pl.pallas_call(
        body,
        grid=grid,
        in_specs=[
            pl.BlockSpec((BN2, 8 * AH), lambda i: (i, 0)),
            pl.BlockSpec((BN2, 8 * AH), lambda i: (i, 0)),
            pl.BlockSpec((BN2, 8), lambda i: (i, 0)),
            pl.BlockSpec((8, 8 * AH), lambda i: (0, 0)),
            pl.BlockSpec((H, 8 * AH, 8), lambda i: (0, 0, 0)),
            pl.BlockSpec((H, 1, 8 * AH), lambda i: (0, 0, 0)),
            pl.BlockSpec((H, 1, 1), lambda i: (0, 0, 0)),
        ],
        out_specs=[
            pl.BlockSpec((BN2, 8), lambda i: (i, 0)),
            pl.BlockSpec((BN2, 8), lambda i: (i, 0)),
        ],
        out_shape=[jax.ShapeDtypeStruct((E8, 8), jnp.float32)] * 2,
    )(q20, q21, elem8, kt, kmat, a1lt, a2r)


B3B = 128
NB3 = 78
EPW3 = NB3 * B3B
NT = NB3 // 2
EXT_BASE = NW * EPW3
NEXT = (E - EXT_BASE) // B3B


NT6 = NB3 // 6


def _b3_body(feat0, feat1, ewr0, ewr1, srcA, dstA, zf, zr,
             pooled0, rsum0, pooled1, rsum1,
             srcb0, dstb0, ewrb0, srcb1, dstb1, ewrb1,
             srcb2, dstb2, ewrb2,
             fbuf0, rsb0, fbuf1, rsb1,
             pool_sp, rs_sp,
             lsem0, lsem1, lsem2, gsem0, gsem1, ssem0, ssem1):
    c = lax.axis_index("c")
    s = lax.axis_index("s")
    wid = s * NC + c
    ebase = wid * EPW3
    rbase = s * RPT

    isets = ((srcb0, dstb0, ewrb0, lsem0),
             (srcb1, dstb1, ewrb1, lsem1),
             (srcb2, dstb2, ewrb2, lsem2))
    dsets = ((fbuf0, rsb0, gsem0, ssem0),
             (fbuf1, rsb1, gsem1, ssem1))

    def issue_ld(b, i, ewr_h):
        srcb, dstb, ewrb, lsem = isets[i]
        base = ebase + b * B3B
        pltpu.async_copy(srcA.at[pl.ds(base, B3B)], srcb, lsem)
        pltpu.async_copy(dstA.at[pl.ds(base, B3B)], dstb, lsem)
        pltpu.async_copy(ewr_h.at[pl.ds(base, B3B)], ewrb, lsem)

    def drain_ld(i, ewr_h):
        srcb, dstb, ewrb, lsem = isets[i]
        pltpu.make_async_copy(srcA.at[pl.ds(0, B3B)], srcb, lsem).wait()
        pltpu.make_async_copy(dstA.at[pl.ds(0, B3B)], dstb, lsem).wait()
        pltpu.make_async_copy(ewr_h.at[pl.ds(0, B3B)], ewrb, lsem).wait()

    def issue_gather(i, d, feat_h):
        fbuf, _, gsem, _ = dsets[d]
        pltpu.async_copy(feat_h.at[isets[i][1]], fbuf, gsem)

    def drain_gather(d):
        fbuf, _, gsem, _ = dsets[d]
        pltpu.make_async_copy(zf.at[pl.ds(0, B3B)], fbuf, gsem).wait()

    def scale(i, d):
        ewrb = isets[i][2]
        fbuf, rsb, _, _ = dsets[d]
        def scl(g, cy):
            ewv = ewrb[pl.ds(g * 16, 16)]
            for l in range(16):
                e = g * 16 + l
                wv = jnp.full((AH,), ewv[l])
                rsb[e, :] = wv
                for j in range(D // AH):
                    fbuf[e, pl.ds(j * AH, AH)] = (
                        fbuf[e, pl.ds(j * AH, AH)] * wv)
            return cy
        lax.fori_loop(0, B3B // 16, scl, 0)

    def issue_scatter(i, d):
        srcb = isets[i][0]
        fbuf, rsb, _, ssem = dsets[d]
        pltpu.async_copy(fbuf, pool_sp.at[srcb], ssem, add=True)
        pltpu.async_copy(rsb, rs_sp.at[srcb], ssem, add=True)

    def drain_scatter(d):
        fbuf, rsb, _, ssem = dsets[d]
        pltpu.make_async_copy(fbuf, pool_sp.at[pl.ds(0, B3B)], ssem).wait()
        pltpu.make_async_copy(rsb, rs_sp.at[pl.ds(0, B3B)], ssem).wait()

    for feat_h, ewr_h, pooled_h, rsum_h in (
            (feat0, ewr0, pooled0, rsum0),
            (feat1, ewr1, pooled1, rsum1)):

        for k in range(RPT // B3B):
            pltpu.sync_copy(zf.at[pl.ds(rbase + k * B3B, B3B)],
                            pool_sp.at[pl.ds(rbase + k * B3B, B3B)])
            pltpu.sync_copy(zr.at[pl.ds(rbase + k * B3B, B3B)],
                            rs_sp.at[pl.ds(rbase + k * B3B, B3B)])
        plsc.subcore_barrier()

        issue_ld(0, 0, ewr_h)
        drain_ld(0, ewr_h)
        issue_gather(0, 0, feat_h)
        issue_ld(1, 1, ewr_h)

        def it(t, carry, feat_h=feat_h, ewr_h=ewr_h):
            for ph in range(6):
                k = 6 * t + ph
                i, i1, i2 = ph % 3, (ph + 1) % 3, (ph + 2) % 3
                d, d1 = ph % 2, (ph + 1) % 2
                if ph == 5:
                    @pl.when(t < NT6 - 1)
                    def _():
                        drain_ld(i1, ewr_h)
                else:
                    drain_ld(i1, ewr_h)
                if ph == 0:
                    @pl.when(t > 0)
                    def _():
                        drain_scatter(d1)
                else:
                    drain_scatter(d1)
                if ph == 5:
                    @pl.when(t < NT6 - 1)
                    def _():
                        issue_gather(i1, d1, feat_h)
                else:
                    issue_gather(i1, d1, feat_h)
                if ph >= 4:
                    @pl.when(t < NT6 - 1)
                    def _():
                        issue_ld(k + 2, i2, ewr_h)
                else:
                    issue_ld(k + 2, i2, ewr_h)
                drain_gather(d)
                scale(i, d)
                issue_scatter(i, d)
            return carry
        lax.fori_loop(0, NT6, it, 0)
        drain_scatter(1)

        @pl.when(wid < NEXT)
        def _(feat_h=feat_h, ewr_h=ewr_h):
            tb = EXT_BASE + wid * B3B
            pltpu.sync_copy(srcA.at[pl.ds(tb, B3B)], srcb0)
            pltpu.sync_copy(dstA.at[pl.ds(tb, B3B)], dstb0)
            pltpu.sync_copy(ewr_h.at[pl.ds(tb, B3B)], ewrb0)
            pltpu.async_copy(feat_h.at[dstb0], fbuf0, gsem0).wait()
            scale(0, 0)
            pltpu.sync_copy(fbuf0, pool_sp.at[srcb0], add=True)
            pltpu.sync_copy(rsb0, rs_sp.at[srcb0], add=True)

        plsc.subcore_barrier()
        pltpu.sync_copy(pool_sp.at[pl.ds(rbase, RPT)],
                        pooled_h.at[c, pl.ds(rbase, RPT)])
        pltpu.sync_copy(rs_sp.at[pl.ds(rbase, RPT)],
                        rsum_h.at[c, pl.ds(rbase, RPT)])
        plsc.subcore_barrier()


def _stage_b3(feat, ewr0, ewr1, src, dst, zf, zr):
    mesh = plsc.VectorSubcoreMesh(core_axis_name="c", subcore_axis_name="s")
    kfn = functools.partial(
        pl.kernel,
        mesh=mesh,
        compiler_params=pltpu.CompilerParams(use_tc_tiling_on_sc=False),
        out_type=[
            jax.ShapeDtypeStruct((NC, N2, D), jnp.float32),
            jax.ShapeDtypeStruct((NC, N2, AH), jnp.float32),
            jax.ShapeDtypeStruct((NC, N2, D), jnp.float32),
            jax.ShapeDtypeStruct((NC, N2, AH), jnp.float32),
        ],
        scratch_types=[
            pltpu.VMEM((B3B,), jnp.int32),
            pltpu.VMEM((B3B,), jnp.int32),
            pltpu.VMEM((B3B,), jnp.float32),
            pltpu.VMEM((B3B,), jnp.int32),
            pltpu.VMEM((B3B,), jnp.int32),
            pltpu.VMEM((B3B,), jnp.float32),
            pltpu.VMEM((B3B,), jnp.int32),
            pltpu.VMEM((B3B,), jnp.int32),
            pltpu.VMEM((B3B,), jnp.float32),
            pltpu.VMEM((B3B, D), jnp.float32),
            pltpu.VMEM((B3B, AH), jnp.float32),
            pltpu.VMEM((B3B, D), jnp.float32),
            pltpu.VMEM((B3B, AH), jnp.float32),
            pltpu.VMEM_SHARED((N2, D), jnp.float32),
            pltpu.VMEM_SHARED((N2, AH), jnp.float32),
            pltpu.SemaphoreType.DMA,
            pltpu.SemaphoreType.DMA,
            pltpu.SemaphoreType.DMA,
            pltpu.SemaphoreType.DMA,
            pltpu.SemaphoreType.DMA,
            pltpu.SemaphoreType.DMA,
            pltpu.SemaphoreType.DMA,
        ],
    )(_b3_body)
    return kfn(feat[0], feat[1], ewr0, ewr1, src, dst, zf, zr)


def _stage_c(p0, r0, p1, r1):
    BN = 1000
    grid = (N // BN,)

    def body(p0_ref, r0_ref, p1_ref, r1_ref, out_ref):
        for h, (p, r) in enumerate(((p0_ref, r0_ref), (p1_ref, r1_ref))):
            pooled = p[0] + p[1]
            rs = r[0, :, 0] + r[1, :, 0] + 1e-10
            out_ref[:, h * D:(h + 1) * D] = pooled / rs[:, None]

    return pl.pallas_call(
        body,
        grid=grid,
        in_specs=[
            pl.BlockSpec((NC, BN, D), lambda i: (0, i, 0)),
            pl.BlockSpec((NC, BN, AH), lambda i: (0, i, 0)),
            pl.BlockSpec((NC, BN, D), lambda i: (0, i, 0)),
            pl.BlockSpec((NC, BN, AH), lambda i: (0, i, 0)),
        ],
        out_specs=pl.BlockSpec((BN, H * D), lambda i: (i, 0)),
        out_shape=jax.ShapeDtypeStruct((N, H * D), jnp.float32),
    )(p0, r0, p1, r1)


def kernel(x, idx, elem, W1, b1, W2, b2, A1, a1, A2, a2):
    A1a = A1[:, :D, :]
    A1b = A1[:, D:2 * D, :]
    feat, ps, pd = _stage_a(x, W1, b1, W2, b2, A1a, a1, A1b)
    src = idx[0]
    dst = idx[1]
    qr0, qr1 = _stage_b1(ps, pd, src, dst)
    kt = jnp.kron(jnp.eye(8, dtype=jnp.float32),
                  jnp.ones((1, AH), jnp.float32))
    a2vt = jnp.tile(A2[:, :, 0], (1, 8))
    kmat = kt.T[None, :, :] * a2vt[:, :, None]
    a1lt = jnp.tile(A1[:, 2 * D, :], (1, 8))[:, None, :]
    a2r = a2[:, :, None]
    ew20, ew21 = _stage_b2(qr0.reshape(E8, 8 * AH), qr1.reshape(E8, 8 * AH),
                           elem.reshape(E8, 8), kt, kmat, a1lt, a2r)
    zf = jnp.zeros((N2, D), jnp.float32)
    zr = jnp.zeros((N2, AH), jnp.float32)
    p0, r0, p1, r1 = _stage_b3(feat, ew20.reshape(E), ew21.reshape(E),
                               src, dst, zf, zr)
    return _stage_c(p0, r0, p1, r1)
